# baseline (speedup 1.0000x reference)
import jax
import jax.numpy as jnp
from jax import lax
from jax.experimental import pallas as pl
from jax.experimental.pallas import tpu as pltpu
from jax.experimental.pallas import tpu_sc as plsc

_EPS = 1e-7
_BN = 2000


def _ln(x, g, b):
    m = jnp.mean(x, axis=-1, keepdims=True)
    v = jnp.mean((x - m) ** 2, axis=-1, keepdims=True)
    return (x - m) / jnp.sqrt(v + 1e-5) * g + b


def _tables(h, tl):
    p = jnp.maximum(h, 0.0) + _EPS
    ex = jnp.exp(tl * p)
    return p * ex, ex


def _enc_body(x_ref, w_ref, b_ref, tl_ref, h_ref, px_ref, xx_ref):
    h = jnp.dot(x_ref[...], w_ref[...], preferred_element_type=jnp.float32)
    h = h + b_ref[...]
    h_ref[...] = h
    px, ex = _tables(h, tl_ref[0])
    px_ref[...] = px
    xx_ref[...] = ex


def _enc(x, enc_W, enc_b, tl):
    N, D = x.shape
    H = enc_W.shape[1]
    row = lambda i: (i, 0)
    full = lambda i: (0, 0)
    return pl.pallas_call(
        _enc_body,
        grid=(N // _BN,),
        in_specs=[
            pl.BlockSpec((_BN, D), row),
            pl.BlockSpec((D, H), full),
            pl.BlockSpec((1, H), full),
            pl.BlockSpec(memory_space=pltpu.SMEM),
        ],
        out_specs=[
            pl.BlockSpec((_BN, H), row),
            pl.BlockSpec((_BN, H), row),
            pl.BlockSpec((_BN, H), row),
        ],
        out_shape=[
            jax.ShapeDtypeStruct((N, H), jnp.float32),
            jax.ShapeDtypeStruct((N, H), jnp.float32),
            jax.ShapeDtypeStruct((N, H), jnp.float32),
        ],
    )(x, enc_W, enc_b.reshape(1, H), tl.reshape(1))


def _sc_agg_call(px, xx, ei4, zeros_nh):
    N, H = px.shape
    NS = 16
    NCH, CB, BLK = ei4.shape[2], ei4.shape[3], ei4.shape[4]
    RPS = 632
    RPSL = N - (NS - 1) * RPS

    def body(px_hbm, x_hbm, ei_hbm, z_hbm, out_hbm,
             table, srcA, dstA, srcB, dstB, rows0, rows1, rows2,
             gsem0, gsem1, gsem2, stsemA, stsemB):
        c = lax.axis_index("c")
        s = lax.axis_index("s")

        @pl.when(s < NS - 1)
        def _z0():
            pltpu.sync_copy(z_hbm.at[pl.ds(pl.multiple_of(s * RPS, 8), RPS)],
                            table.at[pl.ds(pl.multiple_of(s * RPS, 8), RPS)])

        @pl.when(s == NS - 1)
        def _z1():
            pltpu.sync_copy(z_hbm.at[pl.ds((NS - 1) * RPS, RPSL)],
                            table.at[pl.ds((NS - 1) * RPS, RPSL)])

        plsc.subcore_barrier()

        def run(g_hbm):
            pltpu.async_copy(ei_hbm.at[0, s, 0], srcA, stsemA)
            pltpu.async_copy(ei_hbm.at[1, s, 0], dstA, stsemA)

            def do_chunk(k, srcch, dstch, src_nxt, dst_nxt, stsem_cur,
                         stsem_nxt):
                pltpu.make_async_copy(ei_hbm.at[0, s, k], srcch,
                                      stsem_cur).wait()
                pltpu.make_async_copy(ei_hbm.at[1, s, k], dstch,
                                      stsem_cur).wait()

                @pl.when(k + 1 < NCH)
                def _pf():
                    pltpu.async_copy(ei_hbm.at[0, s, k + 1], src_nxt,
                                     stsem_nxt)
                    pltpu.async_copy(ei_hbm.at[1, s, k + 1], dst_nxt,
                                     stsem_nxt)

                pltpu.async_copy(g_hbm.at[srcch.at[0]], rows0, gsem0)
                pltpu.async_copy(g_hbm.at[srcch.at[1]], rows1, gsem1)

                def step(jj, carry):
                    b = 3 * jj
                    pltpu.async_copy(g_hbm.at[srcch.at[b + 2]], rows2, gsem2)
                    pltpu.make_async_copy(g_hbm.at[srcch.at[b]], rows0,
                                          gsem0).wait()
                    pltpu.sync_copy(rows0, table.at[dstch.at[b]], add=True)

                    @pl.when(b + 3 < CB)
                    def _f0():
                        pltpu.async_copy(g_hbm.at[srcch.at[b + 3]], rows0,
                                         gsem0)

                    pltpu.make_async_copy(g_hbm.at[srcch.at[b + 1]], rows1,
                                          gsem1).wait()
                    pltpu.sync_copy(rows1, table.at[dstch.at[b + 1]],
                                    add=True)

                    @pl.when(b + 4 < CB)
                    def _f1():
                        pltpu.async_copy(g_hbm.at[srcch.at[b + 4]], rows1,
                                         gsem1)

                    pltpu.make_async_copy(g_hbm.at[srcch.at[b + 2]], rows2,
                                          gsem2).wait()
                    pltpu.sync_copy(rows2, table.at[dstch.at[b + 2]],
                                    add=True)
                    return carry

                lax.fori_loop(0, CB // 3, step, 0)
                pltpu.make_async_copy(g_hbm.at[srcch.at[CB - 1]], rows0,
                                      gsem0).wait()
                pltpu.sync_copy(rows0, table.at[dstch.at[CB - 1]], add=True)

            def chunk2(kk, carry2):
                do_chunk(2 * kk, srcA, dstA, srcB, dstB, stsemA, stsemB)
                do_chunk(2 * kk + 1, srcB, dstB, srcA, dstA, stsemB, stsemA)
                return carry2

            lax.fori_loop(0, NCH // 2, chunk2, 0)

        @pl.when(c == 0)
        def _c0():
            run(px_hbm)

        @pl.when(c == 1)
        def _c1():
            run(x_hbm)

        plsc.subcore_barrier()

        @pl.when((c == 0) & (s < NS - 1))
        def _w0():
            pltpu.sync_copy(table.at[pl.ds(pl.multiple_of(s * RPS, 8), RPS)],
                            out_hbm.at[0, pl.ds(pl.multiple_of(s * RPS, 8),
                                                RPS)])

        @pl.when((c == 0) & (s == NS - 1))
        def _w0l():
            pltpu.sync_copy(table.at[pl.ds((NS - 1) * RPS, RPSL)],
                            out_hbm.at[0, pl.ds((NS - 1) * RPS, RPSL)])

        @pl.when((c == 1) & (s < NS - 1))
        def _w1():
            pltpu.sync_copy(table.at[pl.ds(pl.multiple_of(s * RPS, 8), RPS)],
                            out_hbm.at[1, pl.ds(pl.multiple_of(s * RPS, 8),
                                                RPS)])

        @pl.when((c == 1) & (s == NS - 1))
        def _w1l():
            pltpu.sync_copy(table.at[pl.ds((NS - 1) * RPS, RPSL)],
                            out_hbm.at[1, pl.ds((NS - 1) * RPS, RPSL)])

    mesh = plsc.VectorSubcoreMesh(core_axis_name="c", subcore_axis_name="s")
    kern = pl.kernel(
        body,
        out_type=jax.ShapeDtypeStruct((2, N, H), jnp.float32),
        mesh=mesh,
        scratch_types=[
            pltpu.VMEM_SHARED((N, H), jnp.float32),
            pltpu.VMEM((CB, BLK), jnp.int32),
            pltpu.VMEM((CB, BLK), jnp.int32),
            pltpu.VMEM((CB, BLK), jnp.int32),
            pltpu.VMEM((CB, BLK), jnp.int32),
            pltpu.VMEM((BLK, H), jnp.float32),
            pltpu.VMEM((BLK, H), jnp.float32),
            pltpu.VMEM((BLK, H), jnp.float32),
            pltpu.SemaphoreType.DMA,
            pltpu.SemaphoreType.DMA,
            pltpu.SemaphoreType.DMA,
            pltpu.SemaphoreType.DMA,
            pltpu.SemaphoreType.DMA,
        ],
    )
    return kern(px, xx, ei4, zeros_nh)


def _make_mlp_body(first):
    def body(s_ref, h_ref, w1_ref, b1_ref, g1_ref, be1_ref, w2_ref, b2_ref,
             dg_ref, db_ref, tl_ref, ho_ref, px_ref, xx_ref):
        h = h_ref[...]
        agg = s_ref[0] / (s_ref[1] + 1e-16)
        o = agg + h
        hh = jnp.dot(o, w1_ref[...], preferred_element_type=jnp.float32)
        hh = _ln(hh + b1_ref[...], g1_ref[...], be1_ref[...])
        z = jnp.maximum(hh, 0.0)
        cc = jnp.dot(z, w2_ref[...], preferred_element_type=jnp.float32)
        cc = cc + b2_ref[...]
        if first:
            hn = cc
        else:
            hn = h + jnp.maximum(_ln(cc, dg_ref[...], db_ref[...]), 0.0)
        ho_ref[...] = hn
        px, ex = _tables(hn, tl_ref[0])
        px_ref[...] = px
        xx_ref[...] = ex

    return body


def _mlp(S, h, W1l, b1l, g1l, be1l, W2l, b2l, dgl, dbl, tnext, first):
    N, H = h.shape
    H2 = W1l.shape[1]
    row = lambda i: (i, 0)
    full = lambda i: (0, 0)
    return pl.pallas_call(
        _make_mlp_body(first),
        grid=(N // _BN,),
        in_specs=[
            pl.BlockSpec((2, _BN, H), lambda i: (0, i, 0)),
            pl.BlockSpec((_BN, H), row),
            pl.BlockSpec((H, H2), full),
            pl.BlockSpec((1, H2), full),
            pl.BlockSpec((1, H2), full),
            pl.BlockSpec((1, H2), full),
            pl.BlockSpec((H2, H), full),
            pl.BlockSpec((1, H), full),
            pl.BlockSpec((1, H), full),
            pl.BlockSpec((1, H), full),
            pl.BlockSpec(memory_space=pltpu.SMEM),
        ],
        out_specs=[
            pl.BlockSpec((_BN, H), row),
            pl.BlockSpec((_BN, H), row),
            pl.BlockSpec((_BN, H), row),
        ],
        out_shape=[
            jax.ShapeDtypeStruct((N, H), jnp.float32),
            jax.ShapeDtypeStruct((N, H), jnp.float32),
            jax.ShapeDtypeStruct((N, H), jnp.float32),
        ],
    )(S, h, W1l, b1l.reshape(1, H2), g1l.reshape(1, H2), be1l.reshape(1, H2),
      W2l, b2l.reshape(1, H), dgl.reshape(1, H), dbl.reshape(1, H),
      tnext.reshape(1))


def _mlp_last_body(s_ref, h_ref, w1_ref, b1_ref, g1_ref, be1_ref, w2_ref,
                   b2_ref, dg_ref, db_ref, dg0_ref, db0_ref, wo_ref, bo_ref,
                   o_ref):
    h = h_ref[...]
    agg = s_ref[0] / (s_ref[1] + 1e-16)
    o = agg + h
    hh = jnp.dot(o, w1_ref[...], preferred_element_type=jnp.float32)
    hh = _ln(hh + b1_ref[...], g1_ref[...], be1_ref[...])
    z = jnp.maximum(hh, 0.0)
    cc = jnp.dot(z, w2_ref[...], preferred_element_type=jnp.float32)
    cc = cc + b2_ref[...]
    hn = h + jnp.maximum(_ln(cc, dg_ref[...], db_ref[...]), 0.0)
    f = jnp.maximum(_ln(hn, dg0_ref[...], db0_ref[...]), 0.0)
    o_ref[...] = jnp.dot(f, wo_ref[...], preferred_element_type=jnp.float32)
    o_ref[...] += bo_ref[...]


def _mlp_last(S, h, W1l, b1l, g1l, be1l, W2l, b2l, dgl, dbl, dg0, db0,
              Wo, bo):
    N, H = h.shape
    H2 = W1l.shape[1]
    CP = Wo.shape[1]
    row = lambda i: (i, 0)
    full = lambda i: (0, 0)
    return pl.pallas_call(
        _mlp_last_body,
        grid=(N // _BN,),
        in_specs=[
            pl.BlockSpec((2, _BN, H), lambda i: (0, i, 0)),
            pl.BlockSpec((_BN, H), row),
            pl.BlockSpec((H, H2), full),
            pl.BlockSpec((1, H2), full),
            pl.BlockSpec((1, H2), full),
            pl.BlockSpec((1, H2), full),
            pl.BlockSpec((H2, H), full),
            pl.BlockSpec((1, H), full),
            pl.BlockSpec((1, H), full),
            pl.BlockSpec((1, H), full),
            pl.BlockSpec((1, H), full),
            pl.BlockSpec((1, H), full),
            pl.BlockSpec((H, CP), full),
            pl.BlockSpec((1, CP), full),
        ],
        out_specs=pl.BlockSpec((_BN, CP), row),
        out_shape=jax.ShapeDtypeStruct((N, CP), jnp.float32),
    )(S, h, W1l, b1l.reshape(1, H2), g1l.reshape(1, H2), be1l.reshape(1, H2),
      W2l, b2l.reshape(1, H), dgl.reshape(1, H), dbl.reshape(1, H),
      dg0.reshape(1, H), db0.reshape(1, H), Wo, bo.reshape(1, CP))


def kernel(x, edge_index, enc_W, enc_b, t, W1, b1, ln_g, ln_b, W2, b2,
           dg, db, out_W, out_b):
    N, D = x.shape
    H = enc_W.shape[1]
    E = edge_index.shape[1]
    C = out_W.shape[1]
    NS, BLK, CB = 16, 80, 25
    ei4 = edge_index.reshape(2, NS, E // (NS * CB * BLK), CB, BLK)
    zeros_nh = jnp.zeros((N, H), jnp.float32)
    CP = 8
    Wo = jnp.pad(out_W, ((0, 0), (0, CP - C)))
    bo = jnp.pad(out_b, (0, CP - C))

    seq = [0, 0, 1, 2, 3]
    h, px, xx = _enc(x, enc_W, enc_b, t[seq[0]])
    for k, l in enumerate(seq):
        S = _sc_agg_call(px, xx, ei4, zeros_nh)
        if k < len(seq) - 1:
            h, px, xx = _mlp(S, h, W1[l], b1[l], ln_g[l], ln_b[l], W2[l],
                             b2[l], dg[l], db[l], t[seq[k + 1]],
                             first=(k == 0))
        else:
            out = _mlp_last(S, h, W1[l], b1[l], ln_g[l], ln_b[l], W2[l],
                            b2[l], dg[l], db[l], dg[0], db[0], Wo, bo)
    return out[:, :C]


# speedup vs baseline: 1.0042x; 1.0042x over previous
import jax
import jax.numpy as jnp
from jax import lax
from jax.experimental import pallas as pl
from jax.experimental.pallas import tpu as pltpu
from jax.experimental.pallas import tpu_sc as plsc

_EPS = 1e-7
_BN = 2000


def _ln(x, g, b):
    m = jnp.mean(x, axis=-1, keepdims=True)
    v = jnp.mean((x - m) ** 2, axis=-1, keepdims=True)
    return (x - m) / jnp.sqrt(v + 1e-5) * g + b


def _tables(h, tl):
    p = jnp.maximum(h, 0.0) + _EPS
    ex = jnp.exp(tl * p)
    return p * ex, ex


def _enc_body(x_ref, w_ref, b_ref, tl_ref, h_ref, px_ref, xx_ref):
    h = jnp.dot(x_ref[...], w_ref[...], preferred_element_type=jnp.float32)
    h = h + b_ref[...]
    h_ref[...] = h
    px, ex = _tables(h, tl_ref[0])
    px_ref[...] = px
    xx_ref[...] = ex


def _enc(x, enc_W, enc_b, tl):
    N, D = x.shape
    H = enc_W.shape[1]
    row = lambda i: (i, 0)
    full = lambda i: (0, 0)
    return pl.pallas_call(
        _enc_body,
        grid=(N // _BN,),
        in_specs=[
            pl.BlockSpec((_BN, D), row),
            pl.BlockSpec((D, H), full),
            pl.BlockSpec((1, H), full),
            pl.BlockSpec(memory_space=pltpu.SMEM),
        ],
        out_specs=[
            pl.BlockSpec((_BN, H), row),
            pl.BlockSpec((_BN, H), row),
            pl.BlockSpec((_BN, H), row),
        ],
        out_shape=[
            jax.ShapeDtypeStruct((N, H), jnp.float32),
            jax.ShapeDtypeStruct((N, H), jnp.float32),
            jax.ShapeDtypeStruct((N, H), jnp.float32),
        ],
    )(x, enc_W, enc_b.reshape(1, H), tl.reshape(1))


def _sc_agg_call(px, xx, ei4, zeros_nh):
    N, H = px.shape
    NS = 16
    NCH, CB, BLK = ei4.shape[2], ei4.shape[3], ei4.shape[4]
    RPS = 632
    RPSL = N - (NS - 1) * RPS

    def body(px_hbm, x_hbm, ei_hbm, z_hbm, out_hbm,
             table, srcA, dstA, srcB, dstB, rows0, rows1, rows2,
             gsem0, gsem1, gsem2, stsemA, stsemB):
        c = lax.axis_index("c")
        s = lax.axis_index("s")

        pltpu.async_copy(ei_hbm.at[0, s, 0], srcA, stsemA)
        pltpu.async_copy(ei_hbm.at[1, s, 0], dstA, stsemA)

        @pl.when(s < NS - 1)
        def _z0():
            pltpu.sync_copy(z_hbm.at[pl.ds(pl.multiple_of(s * RPS, 8), RPS)],
                            table.at[pl.ds(pl.multiple_of(s * RPS, 8), RPS)])

        @pl.when(s == NS - 1)
        def _z1():
            pltpu.sync_copy(z_hbm.at[pl.ds((NS - 1) * RPS, RPSL)],
                            table.at[pl.ds((NS - 1) * RPS, RPSL)])

        plsc.subcore_barrier()

        def run(g_hbm):

            def do_chunk(k, srcch, dstch, src_nxt, dst_nxt, stsem_cur,
                         stsem_nxt):
                pltpu.make_async_copy(ei_hbm.at[0, s, k], srcch,
                                      stsem_cur).wait()
                pltpu.make_async_copy(ei_hbm.at[1, s, k], dstch,
                                      stsem_cur).wait()

                @pl.when(k + 1 < NCH)
                def _pf():
                    pltpu.async_copy(ei_hbm.at[0, s, k + 1], src_nxt,
                                     stsem_nxt)
                    pltpu.async_copy(ei_hbm.at[1, s, k + 1], dst_nxt,
                                     stsem_nxt)

                pltpu.async_copy(g_hbm.at[srcch.at[0]], rows0, gsem0)
                pltpu.async_copy(g_hbm.at[srcch.at[1]], rows1, gsem1)

                def step(jj, carry):
                    b = 3 * jj
                    pltpu.async_copy(g_hbm.at[srcch.at[b + 2]], rows2, gsem2)
                    pltpu.make_async_copy(g_hbm.at[srcch.at[b]], rows0,
                                          gsem0).wait()
                    pltpu.sync_copy(rows0, table.at[dstch.at[b]], add=True)

                    @pl.when(b + 3 < CB)
                    def _f0():
                        pltpu.async_copy(g_hbm.at[srcch.at[b + 3]], rows0,
                                         gsem0)

                    pltpu.make_async_copy(g_hbm.at[srcch.at[b + 1]], rows1,
                                          gsem1).wait()
                    pltpu.sync_copy(rows1, table.at[dstch.at[b + 1]],
                                    add=True)

                    @pl.when(b + 4 < CB)
                    def _f1():
                        pltpu.async_copy(g_hbm.at[srcch.at[b + 4]], rows1,
                                         gsem1)

                    pltpu.make_async_copy(g_hbm.at[srcch.at[b + 2]], rows2,
                                          gsem2).wait()
                    pltpu.sync_copy(rows2, table.at[dstch.at[b + 2]],
                                    add=True)
                    return carry

                lax.fori_loop(0, CB // 3, step, 0)
                pltpu.make_async_copy(g_hbm.at[srcch.at[CB - 1]], rows0,
                                      gsem0).wait()
                pltpu.sync_copy(rows0, table.at[dstch.at[CB - 1]], add=True)

            def chunk2(kk, carry2):
                do_chunk(2 * kk, srcA, dstA, srcB, dstB, stsemA, stsemB)
                do_chunk(2 * kk + 1, srcB, dstB, srcA, dstA, stsemB, stsemA)
                return carry2

            lax.fori_loop(0, NCH // 2, chunk2, 0)

        @pl.when(c == 0)
        def _c0():
            run(px_hbm)

        @pl.when(c == 1)
        def _c1():
            run(x_hbm)

        plsc.subcore_barrier()

        @pl.when((c == 0) & (s < NS - 1))
        def _w0():
            pltpu.sync_copy(table.at[pl.ds(pl.multiple_of(s * RPS, 8), RPS)],
                            out_hbm.at[0, pl.ds(pl.multiple_of(s * RPS, 8),
                                                RPS)])

        @pl.when((c == 0) & (s == NS - 1))
        def _w0l():
            pltpu.sync_copy(table.at[pl.ds((NS - 1) * RPS, RPSL)],
                            out_hbm.at[0, pl.ds((NS - 1) * RPS, RPSL)])

        @pl.when((c == 1) & (s < NS - 1))
        def _w1():
            pltpu.sync_copy(table.at[pl.ds(pl.multiple_of(s * RPS, 8), RPS)],
                            out_hbm.at[1, pl.ds(pl.multiple_of(s * RPS, 8),
                                                RPS)])

        @pl.when((c == 1) & (s == NS - 1))
        def _w1l():
            pltpu.sync_copy(table.at[pl.ds((NS - 1) * RPS, RPSL)],
                            out_hbm.at[1, pl.ds((NS - 1) * RPS, RPSL)])

    mesh = plsc.VectorSubcoreMesh(core_axis_name="c", subcore_axis_name="s")
    kern = pl.kernel(
        body,
        out_type=jax.ShapeDtypeStruct((2, N, H), jnp.float32),
        mesh=mesh,
        scratch_types=[
            pltpu.VMEM_SHARED((N, H), jnp.float32),
            pltpu.VMEM((CB, BLK), jnp.int32),
            pltpu.VMEM((CB, BLK), jnp.int32),
            pltpu.VMEM((CB, BLK), jnp.int32),
            pltpu.VMEM((CB, BLK), jnp.int32),
            pltpu.VMEM((BLK, H), jnp.float32),
            pltpu.VMEM((BLK, H), jnp.float32),
            pltpu.VMEM((BLK, H), jnp.float32),
            pltpu.SemaphoreType.DMA,
            pltpu.SemaphoreType.DMA,
            pltpu.SemaphoreType.DMA,
            pltpu.SemaphoreType.DMA,
            pltpu.SemaphoreType.DMA,
        ],
    )
    return kern(px, xx, ei4, zeros_nh)


def _make_mlp_body(first):
    def body(s_ref, h_ref, w1_ref, b1_ref, g1_ref, be1_ref, w2_ref, b2_ref,
             dg_ref, db_ref, tl_ref, ho_ref, px_ref, xx_ref):
        h = h_ref[...]
        agg = s_ref[0] / (s_ref[1] + 1e-16)
        o = agg + h
        hh = jnp.dot(o, w1_ref[...], preferred_element_type=jnp.float32)
        hh = _ln(hh + b1_ref[...], g1_ref[...], be1_ref[...])
        z = jnp.maximum(hh, 0.0)
        cc = jnp.dot(z, w2_ref[...], preferred_element_type=jnp.float32)
        cc = cc + b2_ref[...]
        if first:
            hn = cc
        else:
            hn = h + jnp.maximum(_ln(cc, dg_ref[...], db_ref[...]), 0.0)
        ho_ref[...] = hn
        px, ex = _tables(hn, tl_ref[0])
        px_ref[...] = px
        xx_ref[...] = ex

    return body


def _mlp(S, h, W1l, b1l, g1l, be1l, W2l, b2l, dgl, dbl, tnext, first):
    N, H = h.shape
    H2 = W1l.shape[1]
    row = lambda i: (i, 0)
    full = lambda i: (0, 0)
    return pl.pallas_call(
        _make_mlp_body(first),
        grid=(N // _BN,),
        in_specs=[
            pl.BlockSpec((2, _BN, H), lambda i: (0, i, 0)),
            pl.BlockSpec((_BN, H), row),
            pl.BlockSpec((H, H2), full),
            pl.BlockSpec((1, H2), full),
            pl.BlockSpec((1, H2), full),
            pl.BlockSpec((1, H2), full),
            pl.BlockSpec((H2, H), full),
            pl.BlockSpec((1, H), full),
            pl.BlockSpec((1, H), full),
            pl.BlockSpec((1, H), full),
            pl.BlockSpec(memory_space=pltpu.SMEM),
        ],
        out_specs=[
            pl.BlockSpec((_BN, H), row),
            pl.BlockSpec((_BN, H), row),
            pl.BlockSpec((_BN, H), row),
        ],
        out_shape=[
            jax.ShapeDtypeStruct((N, H), jnp.float32),
            jax.ShapeDtypeStruct((N, H), jnp.float32),
            jax.ShapeDtypeStruct((N, H), jnp.float32),
        ],
    )(S, h, W1l, b1l.reshape(1, H2), g1l.reshape(1, H2), be1l.reshape(1, H2),
      W2l, b2l.reshape(1, H), dgl.reshape(1, H), dbl.reshape(1, H),
      tnext.reshape(1))


def _mlp_last_body(s_ref, h_ref, w1_ref, b1_ref, g1_ref, be1_ref, w2_ref,
                   b2_ref, dg_ref, db_ref, dg0_ref, db0_ref, wo_ref, bo_ref,
                   o_ref):
    h = h_ref[...]
    agg = s_ref[0] / (s_ref[1] + 1e-16)
    o = agg + h
    hh = jnp.dot(o, w1_ref[...], preferred_element_type=jnp.float32)
    hh = _ln(hh + b1_ref[...], g1_ref[...], be1_ref[...])
    z = jnp.maximum(hh, 0.0)
    cc = jnp.dot(z, w2_ref[...], preferred_element_type=jnp.float32)
    cc = cc + b2_ref[...]
    hn = h + jnp.maximum(_ln(cc, dg_ref[...], db_ref[...]), 0.0)
    f = jnp.maximum(_ln(hn, dg0_ref[...], db0_ref[...]), 0.0)
    o_ref[...] = jnp.dot(f, wo_ref[...], preferred_element_type=jnp.float32)
    o_ref[...] += bo_ref[...]


def _mlp_last(S, h, W1l, b1l, g1l, be1l, W2l, b2l, dgl, dbl, dg0, db0,
              Wo, bo):
    N, H = h.shape
    H2 = W1l.shape[1]
    CP = Wo.shape[1]
    row = lambda i: (i, 0)
    full = lambda i: (0, 0)
    return pl.pallas_call(
        _mlp_last_body,
        grid=(N // _BN,),
        in_specs=[
            pl.BlockSpec((2, _BN, H), lambda i: (0, i, 0)),
            pl.BlockSpec((_BN, H), row),
            pl.BlockSpec((H, H2), full),
            pl.BlockSpec((1, H2), full),
            pl.BlockSpec((1, H2), full),
            pl.BlockSpec((1, H2), full),
            pl.BlockSpec((H2, H), full),
            pl.BlockSpec((1, H), full),
            pl.BlockSpec((1, H), full),
            pl.BlockSpec((1, H), full),
            pl.BlockSpec((1, H), full),
            pl.BlockSpec((1, H), full),
            pl.BlockSpec((H, CP), full),
            pl.BlockSpec((1, CP), full),
        ],
        out_specs=pl.BlockSpec((_BN, CP), row),
        out_shape=jax.ShapeDtypeStruct((N, CP), jnp.float32),
    )(S, h, W1l, b1l.reshape(1, H2), g1l.reshape(1, H2), be1l.reshape(1, H2),
      W2l, b2l.reshape(1, H), dgl.reshape(1, H), dbl.reshape(1, H),
      dg0.reshape(1, H), db0.reshape(1, H), Wo, bo.reshape(1, CP))


def kernel(x, edge_index, enc_W, enc_b, t, W1, b1, ln_g, ln_b, W2, b2,
           dg, db, out_W, out_b):
    N, D = x.shape
    H = enc_W.shape[1]
    E = edge_index.shape[1]
    C = out_W.shape[1]
    NS, BLK, CB = 16, 80, 25
    ei4 = edge_index.reshape(2, NS, E // (NS * CB * BLK), CB, BLK)
    zeros_nh = jnp.zeros((N, H), jnp.float32)
    CP = 8
    Wo = jnp.pad(out_W, ((0, 0), (0, CP - C)))
    bo = jnp.pad(out_b, (0, CP - C))

    seq = [0, 0, 1, 2, 3]
    h, px, xx = _enc(x, enc_W, enc_b, t[seq[0]])
    for k, l in enumerate(seq):
        S = _sc_agg_call(px, xx, ei4, zeros_nh)
        if k < len(seq) - 1:
            h, px, xx = _mlp(S, h, W1[l], b1[l], ln_g[l], ln_b[l], W2[l],
                             b2[l], dg[l], db[l], t[seq[k + 1]],
                             first=(k == 0))
        else:
            out = _mlp_last(S, h, W1[l], b1[l], ln_g[l], ln_b[l], W2[l],
                            b2[l], dg[l], db[l], dg[0], db[0], Wo, bo)
    return out[:, :C]
